# Initial kernel scaffold; baseline (speedup 1.0000x reference)
import jax
import jax.numpy as jnp
from jax.experimental import pallas as pl


def kernel(x, edge_index, W1, b1, W2, b2, W3, b3, g1, be1, g2, be2):
    raise NotImplementedError("write your pallas kernel here")


# speedup vs baseline: 15.4592x; 15.4592x over previous
import functools

import jax
import jax.numpy as jnp
from jax import lax
from jax.experimental import pallas as pl
from jax.experimental.pallas import tpu as pltpu
from jax.experimental.pallas import tpu_sc as plsc

NC = 2
NS = 16
NW = NC * NS
W = 128

_SC_MESH = dict(
    core_axis_name="c", subcore_axis_name="s", num_cores=NC, num_subcores=NS
)


def _deg_kernel(N, E, ib, ops):
    cpw = E // (NW * ops * ib)
    rps = N // NS

    @functools.partial(
        pl.kernel,
        out_type=jax.ShapeDtypeStruct((NC, NS, rps, W), jnp.float32),
        mesh=plsc.VectorSubcoreMesh(**_SC_MESH),
        scratch_types=[
            pltpu.VMEM((ops, ib), jnp.int32),
            pltpu.VMEM((ib, W), jnp.float32),
            pltpu.VMEM_SHARED((N, W), jnp.float32),
        ],
    )
    def deg_kernel(dst_hbm, zeros_hbm, ones_hbm, out_hbm, dst_v, ones_v, acc):
        c = lax.axis_index("c")
        s = lax.axis_index("s")
        w = s * NC + c
        pltpu.sync_copy(ones_hbm, ones_v)
        pltpu.sync_copy(zeros_hbm.at[s], acc.at[pl.ds(s * rps, rps)])
        plsc.subcore_barrier()

        def body(i, _):
            pltpu.sync_copy(dst_hbm.at[w, i], dst_v)
            for j in range(ops):
                pltpu.sync_copy(ones_v, acc.at[dst_v.at[j]], add=True)
            return ()

        lax.fori_loop(0, cpw, body, ())
        plsc.subcore_barrier()
        pltpu.sync_copy(acc.at[pl.ds(s * rps, rps)], out_hbm.at[c, s])

    return deg_kernel


def _agg_kernel(N, E, ib, ops):
    cpw = E // (NW * ops * ib)
    rps = N // NS

    @functools.partial(
        pl.kernel,
        out_type=jax.ShapeDtypeStruct((NC, NS, rps, W), jnp.float32),
        mesh=plsc.VectorSubcoreMesh(**_SC_MESH),
        scratch_types=[
            pltpu.VMEM((ops, ib), jnp.int32),
            pltpu.VMEM((ops, ib), jnp.int32),
            pltpu.VMEM((ib, W), jnp.float32),
            pltpu.VMEM_SHARED((N, W), jnp.float32),
            pltpu.SemaphoreType.DMA,
        ],
    )
    def agg_kernel(
        hs_hbm, src_hbm, dst_hbm, zeros_hbm, out_hbm,
        src_v, dst_v, rows_v, acc, sem,
    ):
        c = lax.axis_index("c")
        s = lax.axis_index("s")
        w = s * NC + c
        pltpu.sync_copy(zeros_hbm.at[s], acc.at[pl.ds(s * rps, rps)])
        plsc.subcore_barrier()

        def body(i, _):
            pltpu.sync_copy(src_hbm.at[w, i], src_v)
            pltpu.sync_copy(dst_hbm.at[w, i], dst_v)
            for j in range(ops):
                pltpu.async_copy(hs_hbm.at[src_v.at[j]], rows_v, sem).wait()
                pltpu.sync_copy(rows_v, acc.at[dst_v.at[j]], add=True)
            return ()

        lax.fori_loop(0, cpw, body, ())
        plsc.subcore_barrier()
        pltpu.sync_copy(acc.at[pl.ds(s * rps, rps)], out_hbm.at[c, s])

    return agg_kernel


def _pad_w(v):
    n, h = v.shape
    return jnp.concatenate([v, jnp.zeros((n, W - h), jnp.float32)], axis=1)


def _tc_pre(x_ref, w_ref, degp_ref, hs_ref, dinv_ref):
    deg = 1.0 + degp_ref[0, :, 0:1] + degp_ref[1, :, 0:1]
    dinv = lax.rsqrt(deg)
    dinv_ref[...] = dinv
    hs = jnp.dot(
        x_ref[...], w_ref[...], preferred_element_type=jnp.float32
    ) * dinv
    hs_ref[...] = _pad_w(hs)


def _bn_relu(pre, g, beta, eps=1e-5):
    mean = jnp.mean(pre, axis=0, keepdims=True)
    var = jnp.mean((pre - mean) * (pre - mean), axis=0, keepdims=True)
    h = g * (pre - mean) / jnp.sqrt(var + eps) + beta
    return jnp.maximum(h, 0.0)


def _tc_mid(sp_ref, hs_ref, dinv_ref, b_ref, g_ref, be_ref, w_ref, out_ref):
    h = b_ref.shape[0]
    dinv = dinv_ref[...]
    pre = dinv * (
        sp_ref[0, :, :h] + sp_ref[1, :, :h] + hs_ref[:, :h]
    ) + b_ref[...]
    hh = _bn_relu(pre, g_ref[...], be_ref[...])
    out_ref[...] = _pad_w(
        jnp.dot(hh, w_ref[...], preferred_element_type=jnp.float32) * dinv
    )


def _tc_mid_nomat(sp_ref, hs_ref, dinv_ref, b_ref, g_ref, be_ref, out_ref):
    h = b_ref.shape[0]
    dinv = dinv_ref[...]
    pre = dinv * (
        sp_ref[0, :, :h] + sp_ref[1, :, :h] + hs_ref[:, :h]
    ) + b_ref[...]
    out_ref[...] = _pad_w(_bn_relu(pre, g_ref[...], be_ref[...]) * dinv)


def _tc_post(sp_ref, hs_ref, dinv_ref, w_ref, b_ref, x_ref, out_ref):
    h = w_ref.shape[0]
    agg = dinv_ref[...] * (
        sp_ref[0, :, :h] + sp_ref[1, :, :h] + hs_ref[:, :h]
    )
    out_ref[...] = (
        jnp.dot(agg, w_ref[...], preferred_element_type=jnp.float32)
        + b_ref[...]
        + x_ref[...]
    )


def kernel(x, edge_index, W1, b1, W2, b2, W3, b3, g1, be1, g2, be2):
    N, D = x.shape
    H = W1.shape[1]
    E = edge_index.shape[1]

    ib = 125
    ops = 16
    cpw = E // (NW * ops * ib)
    rps = N // NS
    assert NW * cpw * ops * ib == E and NS * rps == N

    src = edge_index[0].reshape(NW, cpw, ops, ib)
    dst = edge_index[1].reshape(NW, cpw, ops, ib)
    zeros = jnp.zeros((NS, rps, W), jnp.float32)
    ones = jnp.ones((ib, W), jnp.float32)

    degp = _deg_kernel(N, E, ib, ops)(dst, zeros, ones).reshape(NC, N, W)

    tc_pre = pl.pallas_call(
        _tc_pre,
        out_shape=[
            jax.ShapeDtypeStruct((N, W), jnp.float32),
            jax.ShapeDtypeStruct((N, 1), jnp.float32),
        ],
    )
    hs1, dinv = tc_pre(x, W1, degp)

    agg0 = _agg_kernel(N, E, ib, ops)
    agg = lambda hs: agg0(hs, src, dst, zeros).reshape(NC, N, W)
    sp1 = agg(hs1)

    hs2 = pl.pallas_call(
        _tc_mid, out_shape=jax.ShapeDtypeStruct((N, W), jnp.float32)
    )(sp1, hs1, dinv, b1, g1, be1, W2)

    sp2 = agg(hs2)

    hs3 = pl.pallas_call(
        _tc_mid_nomat, out_shape=jax.ShapeDtypeStruct((N, W), jnp.float32)
    )(sp2, hs2, dinv, b2, g2, be2)

    sp3 = agg(hs3)

    out = pl.pallas_call(
        _tc_post, out_shape=jax.ShapeDtypeStruct((N, D), jnp.float32)
    )(sp3, hs3, dinv, W3, b3, x)

    return out


# speedup vs baseline: 22.2365x; 1.4384x over previous
import functools

import jax
import jax.numpy as jnp
from jax import lax
from jax.experimental import pallas as pl
from jax.experimental.pallas import tpu as pltpu
from jax.experimental.pallas import tpu_sc as plsc

NC = 2
NS = 16
NW = NC * NS

_SC_MESH = dict(
    core_axis_name="c", subcore_axis_name="s", num_cores=NC, num_subcores=NS
)
_SC_PARAMS = pltpu.CompilerParams(use_tc_tiling_on_sc=False)


def _deg_kernel(N, E, ib, ops):
    cpw = E // (NW * ops * ib)
    rps = N // NS

    @functools.partial(
        pl.kernel,
        out_type=jax.ShapeDtypeStruct((NC, NS, rps, 8), jnp.float32),
        mesh=plsc.VectorSubcoreMesh(**_SC_MESH),
        scratch_types=[
            pltpu.VMEM((ops, ib), jnp.int32),
            pltpu.VMEM((ib, 8), jnp.float32),
            pltpu.VMEM_SHARED((N, 8), jnp.float32),
        ],
        compiler_params=_SC_PARAMS,
    )
    def deg_kernel(dst_hbm, zeros_hbm, ones_hbm, out_hbm, dst_v, ones_v, acc):
        c = lax.axis_index("c")
        s = lax.axis_index("s")
        w = s * NC + c
        pltpu.sync_copy(ones_hbm, ones_v)
        pltpu.sync_copy(zeros_hbm.at[s], acc.at[pl.ds(s * rps, rps)])
        plsc.subcore_barrier()

        def body(i, _):
            pltpu.sync_copy(dst_hbm.at[w, i], dst_v)
            for j in range(ops):
                pltpu.sync_copy(ones_v, acc.at[dst_v.at[j]], add=True)
            return ()

        lax.fori_loop(0, cpw, body, ())
        plsc.subcore_barrier()
        pltpu.sync_copy(acc.at[pl.ds(s * rps, rps)], out_hbm.at[c, s])

    return deg_kernel


def _agg_kernel(N, E, H, ib, ops):
    cpw = E // (NW * ops * ib)
    rps = N // NS

    @functools.partial(
        pl.kernel,
        out_type=jax.ShapeDtypeStruct((NC, NS, rps, H), jnp.float32),
        mesh=plsc.VectorSubcoreMesh(**_SC_MESH),
        scratch_types=[
            pltpu.VMEM((ops, ib), jnp.int32),
            pltpu.VMEM((ops, ib), jnp.int32),
            pltpu.VMEM((ib, H), jnp.float32),
            pltpu.VMEM_SHARED((N, H), jnp.float32),
            pltpu.SemaphoreType.DMA,
        ],
        compiler_params=_SC_PARAMS,
    )
    def agg_kernel(
        hs_hbm, src_hbm, dst_hbm, zeros_hbm, out_hbm,
        src_v, dst_v, rows_v, acc, sem,
    ):
        c = lax.axis_index("c")
        s = lax.axis_index("s")
        w = s * NC + c
        pltpu.sync_copy(zeros_hbm.at[s], acc.at[pl.ds(s * rps, rps)])
        plsc.subcore_barrier()

        def body(i, _):
            pltpu.sync_copy(src_hbm.at[w, i], src_v)
            pltpu.sync_copy(dst_hbm.at[w, i], dst_v)
            for j in range(ops):
                pltpu.async_copy(hs_hbm.at[src_v.at[j]], rows_v, sem).wait()
                pltpu.sync_copy(rows_v, acc.at[dst_v.at[j]], add=True)
            return ()

        lax.fori_loop(0, cpw, body, ())
        plsc.subcore_barrier()
        pltpu.sync_copy(acc.at[pl.ds(s * rps, rps)], out_hbm.at[c, s])

    return agg_kernel


def _tc_pre(x_ref, w_ref, degp_ref, hs_ref, dinv_ref):
    deg = 1.0 + degp_ref[0, :, 0:1] + degp_ref[1, :, 0:1]
    dinv = lax.rsqrt(deg)
    dinv_ref[...] = dinv
    hs_ref[...] = jnp.dot(
        x_ref[...], w_ref[...], preferred_element_type=jnp.float32
    ) * dinv


def _bn_relu(pre, g, beta, eps=1e-5):
    mean = jnp.mean(pre, axis=0, keepdims=True)
    var = jnp.mean((pre - mean) * (pre - mean), axis=0, keepdims=True)
    h = g * (pre - mean) / jnp.sqrt(var + eps) + beta
    return jnp.maximum(h, 0.0)


def _tc_mid(sp_ref, hs_ref, dinv_ref, b_ref, g_ref, be_ref, w_ref, out_ref):
    dinv = dinv_ref[...]
    pre = dinv * (sp_ref[0] + sp_ref[1] + hs_ref[...]) + b_ref[...]
    h = _bn_relu(pre, g_ref[...], be_ref[...])
    out_ref[...] = jnp.dot(
        h, w_ref[...], preferred_element_type=jnp.float32
    ) * dinv


def _tc_mid_nomat(sp_ref, hs_ref, dinv_ref, b_ref, g_ref, be_ref, out_ref):
    dinv = dinv_ref[...]
    pre = dinv * (sp_ref[0] + sp_ref[1] + hs_ref[...]) + b_ref[...]
    out_ref[...] = _bn_relu(pre, g_ref[...], be_ref[...]) * dinv


def _tc_post(sp_ref, hs_ref, dinv_ref, w_ref, b_ref, x_ref, out_ref):
    agg = dinv_ref[...] * (sp_ref[0] + sp_ref[1] + hs_ref[...])
    out_ref[...] = (
        jnp.dot(agg, w_ref[...], preferred_element_type=jnp.float32)
        + b_ref[...]
        + x_ref[...]
    )


def kernel(x, edge_index, W1, b1, W2, b2, W3, b3, g1, be1, g2, be2):
    N, D = x.shape
    H = W1.shape[1]
    E = edge_index.shape[1]

    ib = 125
    ops = 16
    cpw = E // (NW * ops * ib)
    rps = N // NS
    assert NW * cpw * ops * ib == E and NS * rps == N

    src = edge_index[0].reshape(NW, cpw, ops, ib)
    dst = edge_index[1].reshape(NW, cpw, ops, ib)
    zerosH = jnp.zeros((NS, rps, H), jnp.float32)
    zeros8 = jnp.zeros((NS, rps, 8), jnp.float32)
    ones8 = jnp.ones((ib, 8), jnp.float32)

    degp = _deg_kernel(N, E, ib, ops)(dst, zeros8, ones8).reshape(NC, N, 8)

    tc_pre = pl.pallas_call(
        _tc_pre,
        out_shape=[
            jax.ShapeDtypeStruct((N, H), jnp.float32),
            jax.ShapeDtypeStruct((N, 1), jnp.float32),
        ],
    )
    hs1, dinv = tc_pre(x, W1, degp)

    agg0 = _agg_kernel(N, E, H, ib, ops)
    agg = lambda hs: agg0(hs, src, dst, zerosH).reshape(NC, N, H)
    sp1 = agg(hs1)

    hs2 = pl.pallas_call(
        _tc_mid, out_shape=jax.ShapeDtypeStruct((N, H), jnp.float32)
    )(sp1, hs1, dinv, b1, g1, be1, W2)

    sp2 = agg(hs2)

    hs3 = pl.pallas_call(
        _tc_mid_nomat, out_shape=jax.ShapeDtypeStruct((N, H), jnp.float32)
    )(sp2, hs2, dinv, b2, g2, be2)

    sp3 = agg(hs3)

    out = pl.pallas_call(
        _tc_post, out_shape=jax.ShapeDtypeStruct((N, D), jnp.float32)
    )(sp3, hs3, dinv, W3, b3, x)

    return out


# speedup vs baseline: 26.0647x; 1.1722x over previous
import functools

import jax
import jax.numpy as jnp
from jax import lax
from jax.experimental import pallas as pl
from jax.experimental.pallas import tpu as pltpu
from jax.experimental.pallas import tpu_sc as plsc

NC = 2
NS = 16
NW = NC * NS

_SC_MESH = dict(
    core_axis_name="c", subcore_axis_name="s", num_cores=NC, num_subcores=NS
)
_SC_PARAMS = pltpu.CompilerParams(use_tc_tiling_on_sc=False)


def _deg_kernel(N, E, ib, ops):
    cpw = E // (NW * ops * ib)
    rps = N // NS

    @functools.partial(
        pl.kernel,
        out_type=jax.ShapeDtypeStruct((NC, NS, rps, 8), jnp.float32),
        mesh=plsc.VectorSubcoreMesh(**_SC_MESH),
        scratch_types=[
            pltpu.VMEM((ops, ib), jnp.int32),
            pltpu.VMEM((ib, 8), jnp.float32),
            pltpu.VMEM_SHARED((N, 8), jnp.float32),
        ],
        compiler_params=_SC_PARAMS,
    )
    def deg_kernel(dst_hbm, zeros_hbm, ones_hbm, out_hbm, dst_v, ones_v, acc):
        c = lax.axis_index("c")
        s = lax.axis_index("s")
        w = s * NC + c
        pltpu.sync_copy(ones_hbm, ones_v)
        pltpu.sync_copy(zeros_hbm.at[s], acc.at[pl.ds(s * rps, rps)])
        plsc.subcore_barrier()

        def body(i, _):
            pltpu.sync_copy(dst_hbm.at[w, i], dst_v)
            for j in range(ops):
                pltpu.sync_copy(ones_v, acc.at[dst_v.at[j]], add=True)
            return ()

        lax.fori_loop(0, cpw, body, ())
        plsc.subcore_barrier()
        pltpu.sync_copy(acc.at[pl.ds(s * rps, rps)], out_hbm.at[c, s])

    return deg_kernel


def _agg_kernel(N, E, H, ib, ops):
    cpw = E // (NW * ops * ib)
    rps = N // NS

    @functools.partial(
        pl.kernel,
        out_type=jax.ShapeDtypeStruct((NC, NS, rps, H), jnp.float32),
        mesh=plsc.VectorSubcoreMesh(**_SC_MESH),
        scratch_types=[
            pltpu.VMEM((ops, ib), jnp.int32),
            pltpu.VMEM((ops, ib), jnp.int32),
            pltpu.VMEM((ib, H), jnp.float32),
            pltpu.VMEM((ib, H), jnp.float32),
            pltpu.VMEM_SHARED((N, H), jnp.float32),
            pltpu.SemaphoreType.DMA((2,)),
            pltpu.SemaphoreType.DMA((2,)),
        ],
        compiler_params=_SC_PARAMS,
    )
    def agg_kernel(
        hs_hbm, src_hbm, dst_hbm, zeros_hbm, out_hbm,
        src_v, dst_v, rows_v0, rows_v1, acc, gsem, ssem,
    ):
        c = lax.axis_index("c")
        s = lax.axis_index("s")
        w = s * NC + c
        rows = (rows_v0, rows_v1)
        pltpu.sync_copy(zeros_hbm.at[s], acc.at[pl.ds(s * rps, rps)])
        plsc.subcore_barrier()

        def body(i, _):
            pltpu.sync_copy(src_hbm.at[w, i], src_v)
            pltpu.sync_copy(dst_hbm.at[w, i], dst_v)
            gathers = [
                pltpu.async_copy(
                    hs_hbm.at[src_v.at[j]], rows[j % 2], gsem.at[j % 2]
                )
                for j in [0]
            ]
            scatters = [None, None]
            for j in range(ops):
                p = j % 2
                gathers[j].wait()
                if j + 1 < ops:
                    q = (j + 1) % 2
                    if scatters[q] is not None:
                        scatters[q].wait()
                    gathers.append(
                        pltpu.async_copy(
                            hs_hbm.at[src_v.at[j + 1]], rows[q], gsem.at[q]
                        )
                    )
                scatters[p] = pltpu.async_copy(
                    rows[p], acc.at[dst_v.at[j]], ssem.at[p], add=True
                )
            scatters[0].wait()
            scatters[1].wait()
            return ()

        lax.fori_loop(0, cpw, body, ())
        plsc.subcore_barrier()
        pltpu.sync_copy(acc.at[pl.ds(s * rps, rps)], out_hbm.at[c, s])

    return agg_kernel


def _tc_pre(x_ref, w_ref, degp_ref, hs_ref, dinv_ref):
    deg = 1.0 + degp_ref[0, :, 0:1] + degp_ref[1, :, 0:1]
    dinv = lax.rsqrt(deg)
    dinv_ref[...] = dinv
    hs_ref[...] = jnp.dot(
        x_ref[...], w_ref[...], preferred_element_type=jnp.float32
    ) * dinv


def _bn_relu(pre, g, beta, eps=1e-5):
    mean = jnp.mean(pre, axis=0, keepdims=True)
    var = jnp.mean((pre - mean) * (pre - mean), axis=0, keepdims=True)
    h = g * (pre - mean) / jnp.sqrt(var + eps) + beta
    return jnp.maximum(h, 0.0)


def _tc_mid(sp_ref, hs_ref, dinv_ref, b_ref, g_ref, be_ref, w_ref, out_ref):
    dinv = dinv_ref[...]
    pre = dinv * (sp_ref[0] + sp_ref[1] + hs_ref[...]) + b_ref[...]
    h = _bn_relu(pre, g_ref[...], be_ref[...])
    out_ref[...] = jnp.dot(
        h, w_ref[...], preferred_element_type=jnp.float32
    ) * dinv


def _tc_mid_nomat(sp_ref, hs_ref, dinv_ref, b_ref, g_ref, be_ref, out_ref):
    dinv = dinv_ref[...]
    pre = dinv * (sp_ref[0] + sp_ref[1] + hs_ref[...]) + b_ref[...]
    out_ref[...] = _bn_relu(pre, g_ref[...], be_ref[...]) * dinv


def _tc_post(sp_ref, hs_ref, dinv_ref, w_ref, b_ref, x_ref, out_ref):
    agg = dinv_ref[...] * (sp_ref[0] + sp_ref[1] + hs_ref[...])
    out_ref[...] = (
        jnp.dot(agg, w_ref[...], preferred_element_type=jnp.float32)
        + b_ref[...]
        + x_ref[...]
    )


def kernel(x, edge_index, W1, b1, W2, b2, W3, b3, g1, be1, g2, be2):
    N, D = x.shape
    H = W1.shape[1]
    E = edge_index.shape[1]

    ib = 125
    ops = 16
    cpw = E // (NW * ops * ib)
    rps = N // NS
    assert NW * cpw * ops * ib == E and NS * rps == N

    src = edge_index[0].reshape(NW, cpw, ops, ib)
    dst = edge_index[1].reshape(NW, cpw, ops, ib)
    zerosH = jnp.zeros((NS, rps, H), jnp.float32)
    zeros8 = jnp.zeros((NS, rps, 8), jnp.float32)
    ones8 = jnp.ones((ib, 8), jnp.float32)

    degp = _deg_kernel(N, E, ib, ops)(dst, zeros8, ones8).reshape(NC, N, 8)

    tc_pre = pl.pallas_call(
        _tc_pre,
        out_shape=[
            jax.ShapeDtypeStruct((N, H), jnp.float32),
            jax.ShapeDtypeStruct((N, 1), jnp.float32),
        ],
    )
    hs1, dinv = tc_pre(x, W1, degp)

    agg0 = _agg_kernel(N, E, H, ib, ops)
    agg = lambda hs: agg0(hs, src, dst, zerosH).reshape(NC, N, H)
    sp1 = agg(hs1)

    hs2 = pl.pallas_call(
        _tc_mid, out_shape=jax.ShapeDtypeStruct((N, H), jnp.float32)
    )(sp1, hs1, dinv, b1, g1, be1, W2)

    sp2 = agg(hs2)

    hs3 = pl.pallas_call(
        _tc_mid_nomat, out_shape=jax.ShapeDtypeStruct((N, H), jnp.float32)
    )(sp2, hs2, dinv, b2, g2, be2)

    sp3 = agg(hs3)

    out = pl.pallas_call(
        _tc_post, out_shape=jax.ShapeDtypeStruct((N, D), jnp.float32)
    )(sp3, hs3, dinv, W3, b3, x)

    return out


# speedup vs baseline: 36.9332x; 1.4170x over previous
import functools

import jax
import jax.numpy as jnp
from jax import lax
from jax.experimental import pallas as pl
from jax.experimental.pallas import tpu as pltpu
from jax.experimental.pallas import tpu_sc as plsc

NC = 2
NS = 16
NW = NC * NS

_SC_MESH = dict(
    core_axis_name="c", subcore_axis_name="s", num_cores=NC, num_subcores=NS
)
_SC_PARAMS = pltpu.CompilerParams(use_tc_tiling_on_sc=False)


def _deg_kernel(N, E, ib, ops):
    cpw = E // (NW * ops * ib)
    rps = N // NS

    @functools.partial(
        pl.kernel,
        out_type=jax.ShapeDtypeStruct((NC, NS, rps, 8), jnp.float32),
        mesh=plsc.VectorSubcoreMesh(**_SC_MESH),
        scratch_types=[
            pltpu.VMEM((ops, ib), jnp.int32),
            pltpu.VMEM((ib, 8), jnp.float32),
            pltpu.VMEM_SHARED((N, 8), jnp.float32),
        ],
        compiler_params=_SC_PARAMS,
    )
    def deg_kernel(dst_hbm, zeros_hbm, ones_hbm, out_hbm, dst_v, ones_v, acc):
        c = lax.axis_index("c")
        s = lax.axis_index("s")
        w = s * NC + c
        pltpu.sync_copy(ones_hbm, ones_v)
        pltpu.sync_copy(zeros_hbm.at[s], acc.at[pl.ds(s * rps, rps)])
        plsc.subcore_barrier()

        def body(i, _):
            pltpu.sync_copy(dst_hbm.at[w, i], dst_v)
            for j in range(ops):
                pltpu.sync_copy(ones_v, acc.at[dst_v.at[j]], add=True)
            return ()

        lax.fori_loop(0, cpw, body, ())
        plsc.subcore_barrier()
        pltpu.sync_copy(acc.at[pl.ds(s * rps, rps)], out_hbm.at[c, s])

    return deg_kernel


def _agg_kernel(N, E, H, ib):
    T = E // (NW * ib)
    rps = N // NS
    NB = 5
    G = 3
    K = T // NB
    assert K * NB == T and K >= 2

    @functools.partial(
        pl.kernel,
        out_type=jax.ShapeDtypeStruct((NC, NS, rps, H), jnp.float32),
        mesh=plsc.VectorSubcoreMesh(**_SC_MESH),
        scratch_types=[
            pltpu.VMEM((T, ib), jnp.int32),
            pltpu.VMEM((T, ib), jnp.int32),
            [pltpu.VMEM((ib, H), jnp.float32) for _ in range(NB)],
            pltpu.VMEM_SHARED((N, H), jnp.float32),
            pltpu.SemaphoreType.DMA((NB,)),
            pltpu.SemaphoreType.DMA((NB,)),
        ],
        compiler_params=_SC_PARAMS,
    )
    def agg_kernel(
        hs_hbm, src_hbm, dst_hbm, zeros_hbm, out_hbm,
        src_v, dst_v, rows, acc, gsem, ssem,
    ):
        c = lax.axis_index("c")
        s = lax.axis_index("s")
        w = s * NC + c
        pltpu.sync_copy(src_hbm.at[w], src_v)
        pltpu.sync_copy(dst_hbm.at[w], dst_v)
        pltpu.sync_copy(zeros_hbm.at[s], acc.at[pl.ds(s * rps, rps)])
        plsc.subcore_barrier()

        def gather(j, b):
            pltpu.async_copy(hs_hbm.at[src_v.at[j]], rows[b], gsem.at[b])

        def gather_wait(b):
            pltpu.make_async_copy(hs_hbm.at[src_v.at[0]], rows[b], gsem.at[b]).wait()

        def scatter(j, b):
            pltpu.async_copy(rows[b], acc.at[dst_v.at[j]], ssem.at[b], add=True)

        def scatter_wait(b):
            pltpu.make_async_copy(rows[b], acc.at[dst_v.at[0]], ssem.at[b]).wait()

        for j in range(G):
            gather(j, j)
        for jj in range(NB):
            if jj + G >= NB:
                scatter_wait((jj + G) % NB)
            gather(jj + G, (jj + G) % NB)
            gather_wait(jj)
            scatter(jj, jj)

        def body(k, _):
            base = k * NB
            for jj in range(NB):
                scatter_wait((jj + G) % NB)
                gather(base + jj + G, (jj + G) % NB)
                gather_wait(jj)
                scatter(base + jj, jj)
            return ()

        lax.fori_loop(1, K - 1, body, ())

        base = (K - 1) * NB
        for jj in range(NB):
            if base + jj + G < T:
                scatter_wait((jj + G) % NB)
                gather(base + jj + G, (jj + G) % NB)
            gather_wait(jj)
            scatter(base + jj, jj)
        for b in range(NB):
            scatter_wait(b)
        plsc.subcore_barrier()
        pltpu.sync_copy(acc.at[pl.ds(s * rps, rps)], out_hbm.at[c, s])

    return agg_kernel


def _tc_pre(x_ref, w_ref, degp_ref, hs_ref, dinv_ref):
    deg = 1.0 + degp_ref[0, :, 0:1] + degp_ref[1, :, 0:1]
    dinv = lax.rsqrt(deg)
    dinv_ref[...] = dinv
    hs_ref[...] = jnp.dot(
        x_ref[...], w_ref[...], preferred_element_type=jnp.float32
    ) * dinv


def _bn_relu(pre, g, beta, eps=1e-5):
    mean = jnp.mean(pre, axis=0, keepdims=True)
    var = jnp.mean((pre - mean) * (pre - mean), axis=0, keepdims=True)
    h = g * (pre - mean) / jnp.sqrt(var + eps) + beta
    return jnp.maximum(h, 0.0)


def _tc_mid(sp_ref, hs_ref, dinv_ref, b_ref, g_ref, be_ref, w_ref, out_ref):
    dinv = dinv_ref[...]
    pre = dinv * (sp_ref[0] + sp_ref[1] + hs_ref[...]) + b_ref[...]
    h = _bn_relu(pre, g_ref[...], be_ref[...])
    out_ref[...] = jnp.dot(
        h, w_ref[...], preferred_element_type=jnp.float32
    ) * dinv


def _tc_mid_nomat(sp_ref, hs_ref, dinv_ref, b_ref, g_ref, be_ref, out_ref):
    dinv = dinv_ref[...]
    pre = dinv * (sp_ref[0] + sp_ref[1] + hs_ref[...]) + b_ref[...]
    out_ref[...] = _bn_relu(pre, g_ref[...], be_ref[...]) * dinv


def _tc_post(sp_ref, hs_ref, dinv_ref, w_ref, b_ref, x_ref, out_ref):
    agg = dinv_ref[...] * (sp_ref[0] + sp_ref[1] + hs_ref[...])
    out_ref[...] = (
        jnp.dot(agg, w_ref[...], preferred_element_type=jnp.float32)
        + b_ref[...]
        + x_ref[...]
    )


def kernel(x, edge_index, W1, b1, W2, b2, W3, b3, g1, be1, g2, be2):
    N, D = x.shape
    H = W1.shape[1]
    E = edge_index.shape[1]

    ib = 125
    ops = 16
    cpw = E // (NW * ops * ib)
    rps = N // NS
    assert NW * cpw * ops * ib == E and NS * rps == N

    src = edge_index[0].reshape(NW, cpw * ops, ib)
    dst = edge_index[1].reshape(NW, cpw * ops, ib)
    dst4 = edge_index[1].reshape(NW, cpw, ops, ib)
    zerosH = jnp.zeros((NS, rps, H), jnp.float32)
    zeros8 = jnp.zeros((NS, rps, 8), jnp.float32)
    ones8 = jnp.ones((ib, 8), jnp.float32)

    degp = _deg_kernel(N, E, ib, ops)(dst4, zeros8, ones8).reshape(NC, N, 8)

    tc_pre = pl.pallas_call(
        _tc_pre,
        out_shape=[
            jax.ShapeDtypeStruct((N, H), jnp.float32),
            jax.ShapeDtypeStruct((N, 1), jnp.float32),
        ],
    )
    hs1, dinv = tc_pre(x, W1, degp)

    agg0 = _agg_kernel(N, E, H, ib)
    agg = lambda hs: agg0(hs, src, dst, zerosH).reshape(NC, N, H)
    sp1 = agg(hs1)

    hs2 = pl.pallas_call(
        _tc_mid, out_shape=jax.ShapeDtypeStruct((N, H), jnp.float32)
    )(sp1, hs1, dinv, b1, g1, be1, W2)

    sp2 = agg(hs2)

    hs3 = pl.pallas_call(
        _tc_mid_nomat, out_shape=jax.ShapeDtypeStruct((N, H), jnp.float32)
    )(sp2, hs2, dinv, b2, g2, be2)

    sp3 = agg(hs3)

    out = pl.pallas_call(
        _tc_post, out_shape=jax.ShapeDtypeStruct((N, D), jnp.float32)
    )(sp3, hs3, dinv, W3, b3, x)

    return out


# speedup vs baseline: 45.3571x; 1.2281x over previous
import functools

import jax
import jax.numpy as jnp
from jax import lax
from jax.experimental import pallas as pl
from jax.experimental.pallas import tpu as pltpu
from jax.experimental.pallas import tpu_sc as plsc

NC = 2
NS = 16
NW = NC * NS

_SC_MESH = dict(
    core_axis_name="c", subcore_axis_name="s", num_cores=NC, num_subcores=NS
)
_SC_PARAMS = pltpu.CompilerParams(use_tc_tiling_on_sc=False)


def _deg_kernel(N, E, ib, ops):
    cpw = E // (NW * ops * ib)
    rps = N // NS

    @functools.partial(
        pl.kernel,
        out_type=jax.ShapeDtypeStruct((NC, NS, rps, 8), jnp.float32),
        mesh=plsc.VectorSubcoreMesh(**_SC_MESH),
        scratch_types=[
            pltpu.VMEM((ops, ib), jnp.int32),
            pltpu.VMEM((ib, 8), jnp.float32),
            pltpu.VMEM_SHARED((N, 8), jnp.float32),
        ],
        compiler_params=_SC_PARAMS,
    )
    def deg_kernel(dst_hbm, zeros_hbm, ones_hbm, out_hbm, dst_v, ones_v, acc):
        c = lax.axis_index("c")
        s = lax.axis_index("s")
        w = s * NC + c
        pltpu.sync_copy(ones_hbm, ones_v)
        pltpu.sync_copy(zeros_hbm.at[s], acc.at[pl.ds(s * rps, rps)])
        plsc.subcore_barrier()

        def body(i, _):
            pltpu.sync_copy(dst_hbm.at[w, i], dst_v)
            for j in range(ops):
                pltpu.sync_copy(ones_v, acc.at[dst_v.at[j]], add=True)
            return ()

        lax.fori_loop(0, cpw, body, ())
        plsc.subcore_barrier()
        pltpu.sync_copy(acc.at[pl.ds(s * rps, rps)], out_hbm.at[c, s])

    return deg_kernel


def _agg_kernel(N, E, H, ib):
    T = E // (NW * ib)
    rps = N // NS
    NB = 5
    G = 3
    K = T // NB
    assert K * NB == T and K >= 2

    @functools.partial(
        pl.kernel,
        out_type=jax.ShapeDtypeStruct((NC, NS, rps, H), jnp.float32),
        mesh=plsc.VectorSubcoreMesh(**_SC_MESH),
        scratch_types=[
            pltpu.VMEM((T, ib), jnp.int32),
            pltpu.VMEM((T, ib), jnp.int32),
            [pltpu.VMEM((ib, H), jnp.float32) for _ in range(NB)],
            pltpu.VMEM_SHARED((N, H), jnp.float32),
            pltpu.SemaphoreType.DMA((NB,)),
            pltpu.SemaphoreType.DMA((NB,)),
        ],
        compiler_params=_SC_PARAMS,
    )
    def agg_kernel(
        hs_hbm, src_hbm, dst_hbm, zeros_hbm, out_hbm,
        src_v, dst_v, rows, acc, gsem, ssem,
    ):
        c = lax.axis_index("c")
        s = lax.axis_index("s")
        w = s * NC + c
        pltpu.sync_copy(src_hbm.at[w], src_v)
        pltpu.sync_copy(dst_hbm.at[w], dst_v)
        pltpu.sync_copy(zeros_hbm.at[s], acc.at[pl.ds(s * rps, rps)])
        plsc.subcore_barrier()

        def gather(j, b):
            pltpu.async_copy(hs_hbm.at[src_v.at[j]], rows[b], gsem.at[b])

        def gather_wait(b):
            pltpu.make_async_copy(hs_hbm.at[src_v.at[0]], rows[b], gsem.at[b]).wait()

        def scatter(j, b):
            pltpu.async_copy(rows[b], acc.at[dst_v.at[j]], ssem.at[b], add=True)

        def scatter_wait(b):
            pltpu.make_async_copy(rows[b], acc.at[dst_v.at[0]], ssem.at[b]).wait()

        for j in range(G):
            gather(j, j)
        for jj in range(NB):
            if jj + G >= NB:
                scatter_wait((jj + G) % NB)
            gather(jj + G, (jj + G) % NB)
            gather_wait(jj)
            scatter(jj, jj)

        def body(k, _):
            base = k * NB
            for jj in range(NB):
                scatter_wait((jj + G) % NB)
                gather(base + jj + G, (jj + G) % NB)
                gather_wait(jj)
                scatter(base + jj, jj)
            return ()

        lax.fori_loop(1, K - 1, body, ())

        base = (K - 1) * NB
        for jj in range(NB):
            if base + jj + G < T:
                scatter_wait((jj + G) % NB)
                gather(base + jj + G, (jj + G) % NB)
            gather_wait(jj)
            scatter(base + jj, jj)
        for b in range(NB):
            scatter_wait(b)
        plsc.subcore_barrier()
        pltpu.sync_copy(acc.at[pl.ds(s * rps, rps)], out_hbm.at[c, s])

    return agg_kernel


def _pack(v):
    n = v.shape[0] // 2
    return jnp.concatenate([v[:n], v[n:]], axis=1)


def _unpack(p):
    h = p.shape[1] // 2
    return jnp.concatenate([p[:, :h], p[:, h:]], axis=0)


def _blockdiag(w):
    h, k = w.shape
    z = jnp.zeros((h, k), jnp.float32)
    return jnp.concatenate(
        [jnp.concatenate([w, z], axis=1), jnp.concatenate([z, w], axis=1)],
        axis=0,
    )


def _tc_pre(x_ref, w_ref, degp_ref, hs_ref, dinv_ref):
    deg = 1.0 + degp_ref[0, :, 0:1] + degp_ref[1, :, 0:1]
    dinv = lax.rsqrt(deg)
    n2 = deg.shape[0] // 2
    h = w_ref.shape[1]
    dinv_p = jnp.concatenate(
        [
            jnp.broadcast_to(dinv[:n2], (n2, h)),
            jnp.broadcast_to(dinv[n2:], (n2, h)),
        ],
        axis=1,
    )
    dinv_ref[...] = dinv_p
    v = jnp.dot(x_ref[...], w_ref[...], preferred_element_type=jnp.float32)
    hs_ref[...] = _pack(v) * dinv_p


def _bn_relu_packed(pre, g, beta, n_nodes, eps=1e-5):
    h = g.shape[0]
    s = jnp.sum(pre, axis=0)
    mean = (s[:h] + s[h:]) / n_nodes
    m2 = jnp.concatenate([mean, mean])
    c = pre - m2
    q = jnp.sum(c * c, axis=0)
    var = (q[:h] + q[h:]) / n_nodes
    inv = jnp.concatenate([g, g]) * lax.rsqrt(
        jnp.concatenate([var, var]) + eps
    )
    return jnp.maximum(c * inv + jnp.concatenate([beta, beta]), 0.0)


def _tc_mid(sp_ref, hs_ref, dinv_ref, b_ref, g_ref, be_ref, w_ref, out_ref):
    dinv_p = dinv_ref[...]
    n_nodes = 2.0 * sp_ref.shape[1]
    pre = dinv_p * (sp_ref[0] + sp_ref[1] + hs_ref[...]) + jnp.concatenate(
        [b_ref[...], b_ref[...]]
    )
    hh = _bn_relu_packed(pre, g_ref[...], be_ref[...], n_nodes)
    out_ref[...] = jnp.dot(
        hh, _blockdiag(w_ref[...]), preferred_element_type=jnp.float32
    ) * dinv_p


def _tc_mid_nomat(sp_ref, hs_ref, dinv_ref, b_ref, g_ref, be_ref, out_ref):
    dinv_p = dinv_ref[...]
    n_nodes = 2.0 * sp_ref.shape[1]
    pre = dinv_p * (sp_ref[0] + sp_ref[1] + hs_ref[...]) + jnp.concatenate(
        [b_ref[...], b_ref[...]]
    )
    out_ref[...] = (
        _bn_relu_packed(pre, g_ref[...], be_ref[...], n_nodes) * dinv_p
    )


def _tc_post(sp_ref, hs_ref, dinv_ref, w_ref, b_ref, x_ref, out_ref):
    agg_p = dinv_ref[...] * (sp_ref[0] + sp_ref[1] + hs_ref[...])
    out_ref[...] = (
        jnp.dot(
            _unpack(agg_p), w_ref[...], preferred_element_type=jnp.float32
        )
        + b_ref[...]
        + x_ref[...]
    )


def kernel(x, edge_index, W1, b1, W2, b2, W3, b3, g1, be1, g2, be2):
    N, D = x.shape
    H = W1.shape[1]
    E = edge_index.shape[1]

    ib = 125
    ops = 16
    cpw = E // (NW * ops * ib)
    rps = N // NS
    assert NW * cpw * ops * ib == E and NS * rps == N

    half = N // 2
    remap = lambda a: jnp.where(a < half, 2 * a, 2 * (a - half) + 1)
    src = remap(edge_index[0]).reshape(NW, cpw * ops, ib)
    dst = remap(edge_index[1]).reshape(NW, cpw * ops, ib)
    dst4 = edge_index[1].reshape(NW, cpw, ops, ib)
    zerosH = jnp.zeros((NS, rps, H), jnp.float32)
    zeros8 = jnp.zeros((NS, rps, 8), jnp.float32)
    ones8 = jnp.ones((ib, 8), jnp.float32)

    degp = _deg_kernel(N, E, ib, ops)(dst4, zeros8, ones8).reshape(NC, N, 8)

    tc_pre = pl.pallas_call(
        _tc_pre,
        out_shape=[
            jax.ShapeDtypeStruct((half, 2 * H), jnp.float32),
            jax.ShapeDtypeStruct((half, 2 * H), jnp.float32),
        ],
    )
    hs1, dinv = tc_pre(x, W1, degp)

    agg0 = _agg_kernel(N, E, H, ib)
    agg = lambda hs: agg0(
        hs.reshape(N, H), src, dst, zerosH
    ).reshape(NC, half, 2 * H)
    sp1 = agg(hs1)

    hs2 = pl.pallas_call(
        _tc_mid, out_shape=jax.ShapeDtypeStruct((half, 2 * H), jnp.float32)
    )(sp1, hs1, dinv, b1, g1, be1, W2)

    sp2 = agg(hs2)

    hs3 = pl.pallas_call(
        _tc_mid_nomat,
        out_shape=jax.ShapeDtypeStruct((half, 2 * H), jnp.float32),
    )(sp2, hs2, dinv, b2, g2, be2)

    sp3 = agg(hs3)

    out = pl.pallas_call(
        _tc_post, out_shape=jax.ShapeDtypeStruct((N, D), jnp.float32)
    )(sp3, hs3, dinv, W3, b3, x)

    return out


# speedup vs baseline: 46.4724x; 1.0246x over previous
import functools

import jax
import jax.numpy as jnp
from jax import lax
from jax.experimental import pallas as pl
from jax.experimental.pallas import tpu as pltpu
from jax.experimental.pallas import tpu_sc as plsc

NC = 2
NS = 16
NW = NC * NS

_SC_MESH = dict(
    core_axis_name="c", subcore_axis_name="s", num_cores=NC, num_subcores=NS
)
_SC_PARAMS = pltpu.CompilerParams(use_tc_tiling_on_sc=False)


def _deg_kernel(N, Epad, ib, ntrash):
    T = Epad // (NW * ib)
    rps = N // NS
    rpz = (N + ntrash) // NS
    half = N // 2

    @functools.partial(
        pl.kernel,
        out_type=[
            jax.ShapeDtypeStruct((NC, NS, rps, 8), jnp.float32),
            jax.ShapeDtypeStruct((NW, T, ib), jnp.int32),
            jax.ShapeDtypeStruct((NW, T, ib), jnp.int32),
        ],
        mesh=plsc.VectorSubcoreMesh(**_SC_MESH),
        scratch_types=[
            pltpu.VMEM((T, ib), jnp.int32),
            pltpu.VMEM((T, ib), jnp.int32),
            pltpu.VMEM((T, ib), jnp.int32),
            pltpu.VMEM((T, ib), jnp.int32),
            pltpu.VMEM((ib, 8), jnp.float32),
            pltpu.VMEM_SHARED((N + ntrash, 8), jnp.float32),
            pltpu.SemaphoreType.DMA,
        ],
        compiler_params=_SC_PARAMS,
    )
    def deg_kernel(
        src_hbm, dst_hbm, zeros_hbm, ones_hbm,
        out_hbm, srcm_hbm, dstm_hbm,
        src_v, dst_v, srcm_v, dstm_v, ones_v, acc, sem,
    ):
        c = lax.axis_index("c")
        s = lax.axis_index("s")
        w = s * NC + c
        pltpu.sync_copy(src_hbm.at[w], src_v)
        pltpu.sync_copy(dst_hbm.at[w], dst_v)
        pltpu.sync_copy(ones_hbm, ones_v)
        pltpu.sync_copy(zeros_hbm.at[s], acc.at[pl.ds(s * rpz, rpz)])
        plsc.subcore_barrier()
        for j in range(T):
            pltpu.async_copy(ones_v, acc.at[dst_v.at[j]], sem, add=True)

        def remap_row(t, _):
            for src, dstm in ((src_v, srcm_v), (dst_v, dstm_v)):
                for k in range(ib // 16):
                    v = src[t, pl.ds(16 * k, 16)]
                    m = jnp.where(
                        v < half,
                        v + v,
                        jnp.where(v < N, v + v - (N - 1), v),
                    )
                    dstm[t, pl.ds(16 * k, 16)] = m
            return ()

        lax.fori_loop(0, T, remap_row, ())
        pltpu.sync_copy(srcm_v, srcm_hbm.at[w])
        pltpu.sync_copy(dstm_v, dstm_hbm.at[w])
        for j in range(T):
            pltpu.make_async_copy(
                ones_v, acc.at[dst_v.at[0]], sem
            ).wait()
        plsc.subcore_barrier()
        pltpu.sync_copy(acc.at[pl.ds(s * rps, rps)], out_hbm.at[c, s])

    return deg_kernel


def _agg_kernel(N, Epad, H, ib, ntrash):
    T = Epad // (NW * ib)
    rps = N // NS
    rpz = (N + ntrash) // NS
    NB = 5
    G = 3
    K = T // NB
    assert K * NB == T and K >= 2

    @functools.partial(
        pl.kernel,
        out_type=jax.ShapeDtypeStruct((NC, NS, rps, H), jnp.float32),
        mesh=plsc.VectorSubcoreMesh(**_SC_MESH),
        scratch_types=[
            pltpu.VMEM((T, ib), jnp.int32),
            pltpu.VMEM((T, ib), jnp.int32),
            [pltpu.VMEM((ib, H), jnp.float32) for _ in range(NB)],
            pltpu.VMEM_SHARED((N + ntrash, H), jnp.float32),
            pltpu.SemaphoreType.DMA((NB,)),
            pltpu.SemaphoreType.DMA((NB,)),
        ],
        compiler_params=_SC_PARAMS,
    )
    def agg_kernel(
        hs_hbm, src_hbm, dst_hbm, zeros_hbm, out_hbm,
        src_v, dst_v, rows, acc, gsem, ssem,
    ):
        c = lax.axis_index("c")
        s = lax.axis_index("s")
        w = s * NC + c
        pltpu.sync_copy(src_hbm.at[w], src_v)
        pltpu.sync_copy(dst_hbm.at[w], dst_v)
        pltpu.sync_copy(zeros_hbm.at[s], acc.at[pl.ds(s * rpz, rpz)])
        plsc.subcore_barrier()

        def gather(j, b):
            pltpu.async_copy(hs_hbm.at[src_v.at[j]], rows[b], gsem.at[b])

        def gather_wait(b):
            pltpu.make_async_copy(hs_hbm.at[src_v.at[0]], rows[b], gsem.at[b]).wait()

        def scatter(j, b):
            pltpu.async_copy(rows[b], acc.at[dst_v.at[j]], ssem.at[b], add=True)

        def scatter_wait(b):
            pltpu.make_async_copy(rows[b], acc.at[dst_v.at[0]], ssem.at[b]).wait()

        for j in range(G):
            gather(j, j)
        for jj in range(NB):
            if jj + G >= NB:
                scatter_wait((jj + G) % NB)
            gather(jj + G, (jj + G) % NB)
            gather_wait(jj)
            scatter(jj, jj)

        def body(k, _):
            base = k * NB
            for jj in range(NB):
                scatter_wait((jj + G) % NB)
                gather(base + jj + G, (jj + G) % NB)
                gather_wait(jj)
                scatter(base + jj, jj)
            return ()

        lax.fori_loop(1, K - 1, body, ())

        base = (K - 1) * NB
        for jj in range(NB):
            if base + jj + G < T:
                scatter_wait((jj + G) % NB)
                gather(base + jj + G, (jj + G) % NB)
            gather_wait(jj)
            scatter(base + jj, jj)
        for b in range(NB):
            scatter_wait(b)
        plsc.subcore_barrier()
        pltpu.sync_copy(acc.at[pl.ds(s * rps, rps)], out_hbm.at[c, s])

    return agg_kernel


def _pack(v):
    n = v.shape[0] // 2
    return jnp.concatenate([v[:n], v[n:]], axis=1)


def _unpack(p):
    h = p.shape[1] // 2
    return jnp.concatenate([p[:, :h], p[:, h:]], axis=0)


def _blockdiag(w):
    h, k = w.shape
    z = jnp.zeros((h, k), jnp.float32)
    return jnp.concatenate(
        [jnp.concatenate([w, z], axis=1), jnp.concatenate([z, w], axis=1)],
        axis=0,
    )


def _tc_pre(x_ref, w_ref, degp_ref, hs_ref, dinv_ref):
    deg = 1.0 + degp_ref[0, :, 0:1] + degp_ref[1, :, 0:1]
    dinv = lax.rsqrt(deg)
    n2 = deg.shape[0] // 2
    h = w_ref.shape[1]
    dinv_p = jnp.concatenate(
        [
            jnp.broadcast_to(dinv[:n2], (n2, h)),
            jnp.broadcast_to(dinv[n2:], (n2, h)),
        ],
        axis=1,
    )
    dinv_ref[...] = dinv_p
    v = jnp.dot(x_ref[...], w_ref[...], preferred_element_type=jnp.float32)
    hs_ref[...] = _pack(v) * dinv_p


def _bn_relu_packed(pre, g, beta, n_nodes, eps=1e-5):
    h = g.shape[0]
    s = jnp.sum(pre, axis=0)
    mean = (s[:h] + s[h:]) / n_nodes
    m2 = jnp.concatenate([mean, mean])
    c = pre - m2
    q = jnp.sum(c * c, axis=0)
    var = (q[:h] + q[h:]) / n_nodes
    inv = jnp.concatenate([g, g]) * lax.rsqrt(
        jnp.concatenate([var, var]) + eps
    )
    return jnp.maximum(c * inv + jnp.concatenate([beta, beta]), 0.0)


def _tc_mid(sp_ref, hs_ref, dinv_ref, b_ref, g_ref, be_ref, w_ref, out_ref):
    dinv_p = dinv_ref[...]
    n_nodes = 2.0 * sp_ref.shape[1]
    pre = dinv_p * (sp_ref[0] + sp_ref[1] + hs_ref[...]) + jnp.concatenate(
        [b_ref[...], b_ref[...]]
    )
    hh = _bn_relu_packed(pre, g_ref[...], be_ref[...], n_nodes)
    out_ref[...] = jnp.dot(
        hh, _blockdiag(w_ref[...]), preferred_element_type=jnp.float32
    ) * dinv_p


def _tc_mid_nomat(sp_ref, hs_ref, dinv_ref, b_ref, g_ref, be_ref, out_ref):
    dinv_p = dinv_ref[...]
    n_nodes = 2.0 * sp_ref.shape[1]
    pre = dinv_p * (sp_ref[0] + sp_ref[1] + hs_ref[...]) + jnp.concatenate(
        [b_ref[...], b_ref[...]]
    )
    out_ref[...] = (
        _bn_relu_packed(pre, g_ref[...], be_ref[...], n_nodes) * dinv_p
    )


def _tc_post(sp_ref, hs_ref, dinv_ref, w_ref, b_ref, x_ref, out_ref):
    agg_p = dinv_ref[...] * (sp_ref[0] + sp_ref[1] + hs_ref[...])
    out_ref[...] = (
        jnp.dot(
            _unpack(agg_p), w_ref[...], preferred_element_type=jnp.float32
        )
        + b_ref[...]
        + x_ref[...]
    )


def kernel(x, edge_index, W1, b1, W2, b2, W3, b3, g1, be1, g2, be2):
    N, D = x.shape
    H = W1.shape[1]
    E = edge_index.shape[1]

    ib = 128
    T = -(-E // (NW * ib * 5)) * 5
    Epad = NW * T * ib
    ntrash = NS * 8
    rps = N // NS
    rpz = (N + ntrash) // NS
    half = N // 2
    assert NS * rps == N and NS * rpz == N + ntrash

    npad = Epad - E
    lane = jnp.arange(npad, dtype=jnp.int32)
    src_n = jnp.concatenate([edge_index[0], lane % N]).reshape(NW, T, ib)
    dst_n = jnp.concatenate(
        [edge_index[1], N + lane % ntrash]
    ).reshape(NW, T, ib)
    zerosH = jnp.zeros((NS, rpz, H), jnp.float32)
    zeros8 = jnp.zeros((NS, rpz, 8), jnp.float32)
    ones8 = jnp.ones((ib, 8), jnp.float32)

    degp, src, dst = _deg_kernel(N, Epad, ib, ntrash)(
        src_n, dst_n, zeros8, ones8
    )
    degp = degp.reshape(NC, N, 8)

    tc_pre = pl.pallas_call(
        _tc_pre,
        out_shape=[
            jax.ShapeDtypeStruct((half, 2 * H), jnp.float32),
            jax.ShapeDtypeStruct((half, 2 * H), jnp.float32),
        ],
    )
    hs1, dinv = tc_pre(x, W1, degp)

    agg0 = _agg_kernel(N, Epad, H, ib, ntrash)
    agg = lambda hs: agg0(
        hs.reshape(N, H), src, dst, zerosH
    ).reshape(NC, half, 2 * H)
    sp1 = agg(hs1)

    hs2 = pl.pallas_call(
        _tc_mid, out_shape=jax.ShapeDtypeStruct((half, 2 * H), jnp.float32)
    )(sp1, hs1, dinv, b1, g1, be1, W2)

    sp2 = agg(hs2)

    hs3 = pl.pallas_call(
        _tc_mid_nomat,
        out_shape=jax.ShapeDtypeStruct((half, 2 * H), jnp.float32),
    )(sp2, hs2, dinv, b2, g2, be2)

    sp3 = agg(hs3)

    out = pl.pallas_call(
        _tc_post, out_shape=jax.ShapeDtypeStruct((N, D), jnp.float32)
    )(sp3, hs3, dinv, W3, b3, x)

    return out


# speedup vs baseline: 49.1355x; 1.0573x over previous
import functools

import jax
import jax.numpy as jnp
from jax import lax
from jax.experimental import pallas as pl
from jax.experimental.pallas import tpu as pltpu
from jax.experimental.pallas import tpu_sc as plsc

NC = 2
NS = 16
NW = NC * NS

_SC_MESH = dict(
    core_axis_name="c", subcore_axis_name="s", num_cores=NC, num_subcores=NS
)
_SC_PARAMS = pltpu.CompilerParams(use_tc_tiling_on_sc=False)


def _deg_kernel(N, Epad, ib, ntrash):
    T = Epad // (NW * ib)
    rps = N // NS
    rpz = (N + ntrash) // NS

    @functools.partial(
        pl.kernel,
        out_type=jax.ShapeDtypeStruct((NC, NS, rps, 8), jnp.float32),
        mesh=plsc.VectorSubcoreMesh(**_SC_MESH),
        scratch_types=[
            pltpu.VMEM((T, ib), jnp.int32),
            pltpu.VMEM((ib, 8), jnp.float32),
            pltpu.VMEM_SHARED((N + ntrash, 8), jnp.float32),
            pltpu.SemaphoreType.DMA,
        ],
        compiler_params=_SC_PARAMS,
    )
    def deg_kernel(dst_hbm, zeros_hbm, ones_hbm, out_hbm,
                   dst_v, ones_v, acc, sem):
        c = lax.axis_index("c")
        s = lax.axis_index("s")
        w = s * NC + c
        pltpu.sync_copy(dst_hbm.at[w], dst_v)
        pltpu.sync_copy(ones_hbm, ones_v)
        pltpu.sync_copy(zeros_hbm.at[s], acc.at[pl.ds(s * rpz, rpz)])
        plsc.subcore_barrier()
        for j in range(T):
            pltpu.async_copy(ones_v, acc.at[dst_v.at[j]], sem, add=True)
        for j in range(T):
            pltpu.make_async_copy(ones_v, acc.at[dst_v.at[0]], sem).wait()
        plsc.subcore_barrier()
        pltpu.sync_copy(acc.at[pl.ds(s * rps, rps)], out_hbm.at[c, s])

    return deg_kernel


def _make_tc_split(N, E, Epad, ntrash):
    half = N // 2
    npad = Epad - E

    def tc_split(ei_ref, srcm_ref, dstm_ref, dstn_ref):
        pad_iota = lax.iota(jnp.int32, npad)
        src_n = jnp.concatenate([ei_ref[0], pad_iota])
        dst_n = jnp.concatenate(
            [ei_ref[1], N + (pad_iota % ntrash)]
        )
        remap = lambda v: jnp.where(
            v < half, v + v, jnp.where(v < N, v + v - (N - 1), v)
        )
        srcm_ref[...] = remap(src_n)
        dstm_ref[...] = remap(dst_n)
        dstn_ref[...] = dst_n

    return tc_split


def _agg_kernel(N, Epad, H, ib, ntrash):
    T = Epad // (NW * ib)
    rps = N // NS
    rpz = (N + ntrash) // NS
    NB = 5
    G = 3
    K = T // NB
    assert K * NB == T and K >= 2

    @functools.partial(
        pl.kernel,
        out_type=jax.ShapeDtypeStruct((NC, NS, rps, H), jnp.float32),
        mesh=plsc.VectorSubcoreMesh(**_SC_MESH),
        scratch_types=[
            pltpu.VMEM((T, ib), jnp.int32),
            pltpu.VMEM((T, ib), jnp.int32),
            [pltpu.VMEM((ib, H), jnp.float32) for _ in range(NB)],
            pltpu.VMEM_SHARED((N + ntrash, H), jnp.float32),
            pltpu.SemaphoreType.DMA((NB,)),
            pltpu.SemaphoreType.DMA((NB,)),
        ],
        compiler_params=_SC_PARAMS,
    )
    def agg_kernel(
        hs_hbm, src_hbm, dst_hbm, zeros_hbm, out_hbm,
        src_v, dst_v, rows, acc, gsem, ssem,
    ):
        c = lax.axis_index("c")
        s = lax.axis_index("s")
        w = s * NC + c
        pltpu.sync_copy(src_hbm.at[w], src_v)
        pltpu.sync_copy(dst_hbm.at[w], dst_v)
        pltpu.sync_copy(zeros_hbm.at[s], acc.at[pl.ds(s * rpz, rpz)])
        plsc.subcore_barrier()

        def gather(j, b):
            pltpu.async_copy(hs_hbm.at[src_v.at[j]], rows[b], gsem.at[b])

        def gather_wait(b):
            pltpu.make_async_copy(hs_hbm.at[src_v.at[0]], rows[b], gsem.at[b]).wait()

        def scatter(j, b):
            pltpu.async_copy(rows[b], acc.at[dst_v.at[j]], ssem.at[b], add=True)

        def scatter_wait(b):
            pltpu.make_async_copy(rows[b], acc.at[dst_v.at[0]], ssem.at[b]).wait()

        for j in range(G):
            gather(j, j)
        for jj in range(NB):
            if jj + G >= NB:
                scatter_wait((jj + G) % NB)
            gather(jj + G, (jj + G) % NB)
            gather_wait(jj)
            scatter(jj, jj)

        def body(k, _):
            base = k * NB
            for jj in range(NB):
                scatter_wait((jj + G) % NB)
                gather(base + jj + G, (jj + G) % NB)
                gather_wait(jj)
                scatter(base + jj, jj)
            return ()

        lax.fori_loop(1, K - 1, body, ())

        base = (K - 1) * NB
        for jj in range(NB):
            if base + jj + G < T:
                scatter_wait((jj + G) % NB)
                gather(base + jj + G, (jj + G) % NB)
            gather_wait(jj)
            scatter(base + jj, jj)
        for b in range(NB):
            scatter_wait(b)
        plsc.subcore_barrier()
        pltpu.sync_copy(acc.at[pl.ds(s * rps, rps)], out_hbm.at[c, s])

    return agg_kernel


def _pack(v):
    n = v.shape[0] // 2
    return jnp.concatenate([v[:n], v[n:]], axis=1)


def _unpack(p):
    h = p.shape[1] // 2
    return jnp.concatenate([p[:, :h], p[:, h:]], axis=0)


def _blockdiag(w):
    h, k = w.shape
    z = jnp.zeros((h, k), jnp.float32)
    return jnp.concatenate(
        [jnp.concatenate([w, z], axis=1), jnp.concatenate([z, w], axis=1)],
        axis=0,
    )


def _tc_pre(x_ref, w_ref, degp_ref, hs_ref, dinv_ref):
    deg = 1.0 + degp_ref[0, :, 0:1] + degp_ref[1, :, 0:1]
    dinv = lax.rsqrt(deg)
    n2 = deg.shape[0] // 2
    h = w_ref.shape[1]
    dinv_p = jnp.concatenate(
        [
            jnp.broadcast_to(dinv[:n2], (n2, h)),
            jnp.broadcast_to(dinv[n2:], (n2, h)),
        ],
        axis=1,
    )
    dinv_ref[...] = dinv_p
    v = jnp.dot(x_ref[...], w_ref[...], preferred_element_type=jnp.float32)
    hs_ref[...] = _pack(v) * dinv_p


def _bn_relu_packed(pre, g, beta, n_nodes, eps=1e-5):
    h = g.shape[0]
    s = jnp.sum(pre, axis=0)
    mean = (s[:h] + s[h:]) / n_nodes
    m2 = jnp.concatenate([mean, mean])
    c = pre - m2
    q = jnp.sum(c * c, axis=0)
    var = (q[:h] + q[h:]) / n_nodes
    inv = jnp.concatenate([g, g]) * lax.rsqrt(
        jnp.concatenate([var, var]) + eps
    )
    return jnp.maximum(c * inv + jnp.concatenate([beta, beta]), 0.0)


def _tc_mid(sp_ref, hs_ref, dinv_ref, b_ref, g_ref, be_ref, w_ref, out_ref):
    dinv_p = dinv_ref[...]
    n_nodes = 2.0 * sp_ref.shape[1]
    pre = dinv_p * (sp_ref[0] + sp_ref[1] + hs_ref[...]) + jnp.concatenate(
        [b_ref[...], b_ref[...]]
    )
    hh = _bn_relu_packed(pre, g_ref[...], be_ref[...], n_nodes)
    out_ref[...] = jnp.dot(
        hh, _blockdiag(w_ref[...]), preferred_element_type=jnp.float32
    ) * dinv_p


def _tc_mid_nomat(sp_ref, hs_ref, dinv_ref, b_ref, g_ref, be_ref, out_ref):
    dinv_p = dinv_ref[...]
    n_nodes = 2.0 * sp_ref.shape[1]
    pre = dinv_p * (sp_ref[0] + sp_ref[1] + hs_ref[...]) + jnp.concatenate(
        [b_ref[...], b_ref[...]]
    )
    out_ref[...] = (
        _bn_relu_packed(pre, g_ref[...], be_ref[...], n_nodes) * dinv_p
    )


def _tc_post(sp_ref, hs_ref, dinv_ref, w_ref, b_ref, x_ref, out_ref):
    agg_p = dinv_ref[...] * (sp_ref[0] + sp_ref[1] + hs_ref[...])
    out_ref[...] = (
        jnp.dot(
            _unpack(agg_p), w_ref[...], preferred_element_type=jnp.float32
        )
        + b_ref[...]
        + x_ref[...]
    )


def kernel(x, edge_index, W1, b1, W2, b2, W3, b3, g1, be1, g2, be2):
    N, D = x.shape
    H = W1.shape[1]
    E = edge_index.shape[1]

    ib = 128
    T = -(-E // (NW * ib * 5)) * 5
    Epad = NW * T * ib
    ntrash = NS * 8
    rps = N // NS
    rpz = (N + ntrash) // NS
    half = N // 2
    assert NS * rps == N and NS * rpz == N + ntrash

    zerosH = jnp.zeros((NS, rpz, H), jnp.float32)
    zeros8 = jnp.zeros((NS, rpz, 8), jnp.float32)
    ones8 = jnp.ones((ib, 8), jnp.float32)

    srcm1, dstm1, dstn1 = pl.pallas_call(
        _make_tc_split(N, E, Epad, ntrash),
        out_shape=[jax.ShapeDtypeStruct((Epad,), jnp.int32)] * 3,
    )(edge_index)
    src = srcm1.reshape(NW, T, ib)
    dst = dstm1.reshape(NW, T, ib)
    dst_n = dstn1.reshape(NW, T, ib)

    degp = _deg_kernel(N, Epad, ib, ntrash)(dst_n, zeros8, ones8)
    degp = degp.reshape(NC, N, 8)

    tc_pre = pl.pallas_call(
        _tc_pre,
        out_shape=[
            jax.ShapeDtypeStruct((half, 2 * H), jnp.float32),
            jax.ShapeDtypeStruct((half, 2 * H), jnp.float32),
        ],
    )
    hs1, dinv = tc_pre(x, W1, degp)

    agg0 = _agg_kernel(N, Epad, H, ib, ntrash)
    agg = lambda hs: agg0(
        hs.reshape(N, H), src, dst, zerosH
    ).reshape(NC, half, 2 * H)
    sp1 = agg(hs1)

    hs2 = pl.pallas_call(
        _tc_mid, out_shape=jax.ShapeDtypeStruct((half, 2 * H), jnp.float32)
    )(sp1, hs1, dinv, b1, g1, be1, W2)

    sp2 = agg(hs2)

    hs3 = pl.pallas_call(
        _tc_mid_nomat,
        out_shape=jax.ShapeDtypeStruct((half, 2 * H), jnp.float32),
    )(sp2, hs2, dinv, b2, g2, be2)

    sp3 = agg(hs3)

    out = pl.pallas_call(
        _tc_post, out_shape=jax.ShapeDtypeStruct((N, D), jnp.float32)
    )(sp3, hs3, dinv, W3, b3, x)

    return out


# speedup vs baseline: 49.3353x; 1.0041x over previous
import functools

import jax
import jax.numpy as jnp
from jax import lax
from jax.experimental import pallas as pl
from jax.experimental.pallas import tpu as pltpu
from jax.experimental.pallas import tpu_sc as plsc

NC = 2
NS = 16
NW = NC * NS

_SC_MESH = dict(
    core_axis_name="c", subcore_axis_name="s", num_cores=NC, num_subcores=NS
)
_SC_PARAMS = pltpu.CompilerParams(use_tc_tiling_on_sc=False)


def _deg_kernel(N, Epad, ib, ntrash):
    T = Epad // (NW * ib)
    rps = N // NS
    rpz = (N + ntrash) // NS

    @functools.partial(
        pl.kernel,
        out_type=jax.ShapeDtypeStruct((NC, NS, rps, 8), jnp.float32),
        mesh=plsc.VectorSubcoreMesh(**_SC_MESH),
        scratch_types=[
            pltpu.VMEM((T, ib), jnp.int32),
            pltpu.VMEM((ib, 8), jnp.float32),
            pltpu.VMEM_SHARED((N + ntrash, 8), jnp.float32),
            pltpu.SemaphoreType.DMA,
        ],
        compiler_params=_SC_PARAMS,
    )
    def deg_kernel(dst_hbm, zeros_hbm, ones_hbm, out_hbm,
                   dst_v, ones_v, acc, sem):
        c = lax.axis_index("c")
        s = lax.axis_index("s")
        w = s * NC + c
        pltpu.sync_copy(dst_hbm.at[w], dst_v)
        pltpu.sync_copy(ones_hbm, ones_v)
        pltpu.sync_copy(zeros_hbm.at[s], acc.at[pl.ds(s * rpz, rpz)])
        plsc.subcore_barrier()
        for j in range(T):
            pltpu.async_copy(ones_v, acc.at[dst_v.at[j]], sem, add=True)
        for j in range(T):
            pltpu.make_async_copy(ones_v, acc.at[dst_v.at[0]], sem).wait()
        plsc.subcore_barrier()
        pltpu.sync_copy(acc.at[pl.ds(s * rps, rps)], out_hbm.at[c, s])

    return deg_kernel


def _make_tc_split(N, E, Epad, ntrash):
    half = N // 2
    npad = Epad - E

    def tc_split(ei_ref, srcm_ref, dstm_ref, dstn_ref):
        pad_iota = lax.iota(jnp.int32, npad)
        src_n = jnp.concatenate([ei_ref[0], pad_iota])
        dst_n = jnp.concatenate(
            [ei_ref[1], N + (pad_iota % ntrash)]
        )
        remap = lambda v: jnp.where(
            v < half, v + v, jnp.where(v < N, v + v - (N - 1), v)
        )
        srcm_ref[...] = remap(src_n)
        dstm_ref[...] = remap(dst_n)
        dstn_ref[...] = dst_n

    return tc_split


def _agg_kernel(N, Epad, H, ib, ntrash):
    T = Epad // (NW * ib)
    rps = N // NS
    rpz = (N + ntrash) // NS
    NB = 5
    G = 3
    K = T // NB
    assert K * NB == T and K >= 2

    @functools.partial(
        pl.kernel,
        out_type=jax.ShapeDtypeStruct((NC, NS, rps, H), jnp.float32),
        mesh=plsc.VectorSubcoreMesh(**_SC_MESH),
        scratch_types=[
            pltpu.VMEM((T, ib), jnp.int32),
            pltpu.VMEM((T, ib), jnp.int32),
            [pltpu.VMEM((ib, H), jnp.float32) for _ in range(NB)],
            pltpu.VMEM_SHARED((N + ntrash, H), jnp.float32),
            pltpu.SemaphoreType.DMA((NB,)),
            pltpu.SemaphoreType.DMA((NB,)),
        ],
        compiler_params=_SC_PARAMS,
    )
    def agg_kernel(
        hs_hbm, src_hbm, dst_hbm, zeros_hbm, out_hbm,
        src_v, dst_v, rows, acc, gsem, ssem,
    ):
        c = lax.axis_index("c")
        s = lax.axis_index("s")
        w = s * NC + c
        pltpu.sync_copy(src_hbm.at[w], src_v)
        pltpu.sync_copy(dst_hbm.at[w], dst_v)
        tps = ntrash // NS

        @pl.when(c == 0)
        def _():
            pltpu.sync_copy(
                hs_hbm.at[pl.ds(s * rps, rps)], acc.at[pl.ds(s * rps, rps)]
            )
            pltpu.sync_copy(
                zeros_hbm.at[s, pl.ds(0, tps)],
                acc.at[pl.ds(N + s * tps, tps)],
            )

        @pl.when(c != 0)
        def _():
            pltpu.sync_copy(zeros_hbm.at[s], acc.at[pl.ds(s * rpz, rpz)])

        plsc.subcore_barrier()

        def gather(j, b):
            pltpu.async_copy(hs_hbm.at[src_v.at[j]], rows[b], gsem.at[b])

        def gather_wait(b):
            pltpu.make_async_copy(hs_hbm.at[src_v.at[0]], rows[b], gsem.at[b]).wait()

        def scatter(j, b):
            pltpu.async_copy(rows[b], acc.at[dst_v.at[j]], ssem.at[b], add=True)

        def scatter_wait(b):
            pltpu.make_async_copy(rows[b], acc.at[dst_v.at[0]], ssem.at[b]).wait()

        for j in range(G):
            gather(j, j)
        for jj in range(NB):
            if jj + G >= NB:
                scatter_wait((jj + G) % NB)
            gather(jj + G, (jj + G) % NB)
            gather_wait(jj)
            scatter(jj, jj)

        def body(k, _):
            base = k * NB
            for jj in range(NB):
                scatter_wait((jj + G) % NB)
                gather(base + jj + G, (jj + G) % NB)
                gather_wait(jj)
                scatter(base + jj, jj)
            return ()

        lax.fori_loop(1, K - 1, body, ())

        base = (K - 1) * NB
        for jj in range(NB):
            if base + jj + G < T:
                scatter_wait((jj + G) % NB)
                gather(base + jj + G, (jj + G) % NB)
            gather_wait(jj)
            scatter(base + jj, jj)
        for b in range(NB):
            scatter_wait(b)
        plsc.subcore_barrier()
        pltpu.sync_copy(acc.at[pl.ds(s * rps, rps)], out_hbm.at[c, s])

    return agg_kernel


def _pack(v):
    n = v.shape[0] // 2
    return jnp.concatenate([v[:n], v[n:]], axis=1)


def _unpack(p):
    h = p.shape[1] // 2
    return jnp.concatenate([p[:, :h], p[:, h:]], axis=0)


def _blockdiag(w):
    h, k = w.shape
    z = jnp.zeros((h, k), jnp.float32)
    return jnp.concatenate(
        [jnp.concatenate([w, z], axis=1), jnp.concatenate([z, w], axis=1)],
        axis=0,
    )


def _tc_pre(x_ref, w_ref, degp_ref, hs_ref, dinv_ref):
    deg = 1.0 + degp_ref[0, :, 0:1] + degp_ref[1, :, 0:1]
    dinv = lax.rsqrt(deg)
    n2 = deg.shape[0] // 2
    h = w_ref.shape[1]
    dinv_p = jnp.concatenate(
        [
            jnp.broadcast_to(dinv[:n2], (n2, h)),
            jnp.broadcast_to(dinv[n2:], (n2, h)),
        ],
        axis=1,
    )
    dinv_ref[...] = dinv_p
    v = jnp.dot(x_ref[...], w_ref[...], preferred_element_type=jnp.float32)
    hs_ref[...] = _pack(v) * dinv_p


def _bn_relu_packed(pre, g, beta, n_nodes, eps=1e-5):
    h = g.shape[0]
    s = jnp.sum(pre, axis=0)
    mean = (s[:h] + s[h:]) / n_nodes
    m2 = jnp.concatenate([mean, mean])
    c = pre - m2
    q = jnp.sum(c * c, axis=0)
    var = (q[:h] + q[h:]) / n_nodes
    inv = jnp.concatenate([g, g]) * lax.rsqrt(
        jnp.concatenate([var, var]) + eps
    )
    return jnp.maximum(c * inv + jnp.concatenate([beta, beta]), 0.0)


def _tc_mid(sp_ref, dinv_ref, b_ref, g_ref, be_ref, w_ref, out_ref):
    dinv_p = dinv_ref[...]
    n_nodes = 2.0 * sp_ref.shape[1]
    pre = dinv_p * (sp_ref[0] + sp_ref[1]) + jnp.concatenate(
        [b_ref[...], b_ref[...]]
    )
    hh = _bn_relu_packed(pre, g_ref[...], be_ref[...], n_nodes)
    out_ref[...] = jnp.dot(
        hh, _blockdiag(w_ref[...]), preferred_element_type=jnp.float32
    ) * dinv_p


def _tc_mid_nomat(sp_ref, dinv_ref, b_ref, g_ref, be_ref, out_ref):
    dinv_p = dinv_ref[...]
    n_nodes = 2.0 * sp_ref.shape[1]
    pre = dinv_p * (sp_ref[0] + sp_ref[1]) + jnp.concatenate(
        [b_ref[...], b_ref[...]]
    )
    out_ref[...] = (
        _bn_relu_packed(pre, g_ref[...], be_ref[...], n_nodes) * dinv_p
    )


def _tc_post(sp_ref, dinv_ref, w_ref, b_ref, x_ref, out_ref):
    agg_p = dinv_ref[...] * (sp_ref[0] + sp_ref[1])
    out_ref[...] = (
        jnp.dot(
            _unpack(agg_p), w_ref[...], preferred_element_type=jnp.float32
        )
        + b_ref[...]
        + x_ref[...]
    )


def kernel(x, edge_index, W1, b1, W2, b2, W3, b3, g1, be1, g2, be2):
    N, D = x.shape
    H = W1.shape[1]
    E = edge_index.shape[1]

    ib = 128
    T = -(-E // (NW * ib * 5)) * 5
    Epad = NW * T * ib
    ntrash = NS * 8
    rps = N // NS
    rpz = (N + ntrash) // NS
    half = N // 2
    assert NS * rps == N and NS * rpz == N + ntrash

    zerosH = jnp.zeros((NS, rpz, H), jnp.float32)
    zeros8 = jnp.zeros((NS, rpz, 8), jnp.float32)
    ones8 = jnp.ones((ib, 8), jnp.float32)

    srcm1, dstm1, dstn1 = pl.pallas_call(
        _make_tc_split(N, E, Epad, ntrash),
        out_shape=[jax.ShapeDtypeStruct((Epad,), jnp.int32)] * 3,
    )(edge_index)
    src = srcm1.reshape(NW, T, ib)
    dst = dstm1.reshape(NW, T, ib)
    dst_n = dstn1.reshape(NW, T, ib)

    degp = _deg_kernel(N, Epad, ib, ntrash)(dst_n, zeros8, ones8)
    degp = degp.reshape(NC, N, 8)

    tc_pre = pl.pallas_call(
        _tc_pre,
        out_shape=[
            jax.ShapeDtypeStruct((half, 2 * H), jnp.float32),
            jax.ShapeDtypeStruct((half, 2 * H), jnp.float32),
        ],
    )
    hs1, dinv = tc_pre(x, W1, degp)

    agg0 = _agg_kernel(N, Epad, H, ib, ntrash)
    agg = lambda hs: agg0(
        hs.reshape(N, H), src, dst, zerosH
    ).reshape(NC, half, 2 * H)
    sp1 = agg(hs1)

    hs2 = pl.pallas_call(
        _tc_mid, out_shape=jax.ShapeDtypeStruct((half, 2 * H), jnp.float32)
    )(sp1, dinv, b1, g1, be1, W2)

    sp2 = agg(hs2)

    hs3 = pl.pallas_call(
        _tc_mid_nomat,
        out_shape=jax.ShapeDtypeStruct((half, 2 * H), jnp.float32),
    )(sp2, dinv, b2, g2, be2)

    sp3 = agg(hs3)

    out = pl.pallas_call(
        _tc_post, out_shape=jax.ShapeDtypeStruct((N, D), jnp.float32)
    )(sp3, dinv, W3, b3, x)

    return out
